# Initial kernel scaffold; baseline (speedup 1.0000x reference)
import jax
import jax.numpy as jnp
from jax.experimental import pallas as pl


def kernel(x, edge_index, edge_attr, batch, lin0_W, lin0_b, msg_W, msg_b, gru_Wi, gru_Wh, gru_bi, gru_bh, lstm_Wi, lstm_Wh, lstm_b, mlp_W1, mlp_b1, mlp_W2, mlp_b2, mlp_W3, mlp_b3):
    raise NotImplementedError("write your pallas kernel here")


# speedup vs baseline: 2.7621x; 2.7621x over previous
import functools

import jax
import jax.numpy as jnp
from jax import lax
from jax.experimental import pallas as pl
from jax.experimental.pallas import tpu as pltpu
from jax.experimental.pallas import tpu_sc as plsc

N = 10000
E = 160000
NF = 128
ED = 16
D = 128
B = 64

NP = 10240
NC = 2
NS = 16
NW = NC * NS
CH = 128
CPW = 40
EP = NW * CPW * CH
TRASH = NP - N
EB = 1280
ROWB = 2048

_PREC = lax.Precision.HIGHEST


def _dot(a, b):
    return jnp.dot(a, b, precision=_PREC, preferred_element_type=jnp.float32)


def _lin0_body(x_ref, w0_ref, b0_ref, wt_ref, out_ref, p_ref):
    h = jnp.maximum(_dot(x_ref[...], w0_ref[...]) + b0_ref[...], 0.0)
    out_ref[...] = h
    p_ref[...] = _dot(h, wt_ref[...])


def _lin0(xp, w0, b0, wt):
    return pl.pallas_call(
        _lin0_body,
        grid=(NP // ROWB,),
        in_specs=[
            pl.BlockSpec((ROWB, NF), lambda i: (i, 0)),
            pl.BlockSpec((NF, D), lambda i: (0, 0)),
            pl.BlockSpec((1, D), lambda i: (0, 0)),
            pl.BlockSpec((D, D), lambda i: (0, 0)),
        ],
        out_specs=[
            pl.BlockSpec((ROWB, D), lambda i: (i, 0)),
            pl.BlockSpec((ROWB, D), lambda i: (i, 0)),
        ],
        out_shape=[
            jax.ShapeDtypeStruct((NP, D), jnp.float32),
            jax.ShapeDtypeStruct((NP, D), jnp.float32),
        ],
    )(xp, w0, b0, wt)


def _eterm_body(ea_ref, wb_ref, b_ref, out_ref):
    out_ref[...] = _dot(ea_ref[...], wb_ref[...]) + b_ref[...]


def _eterm(edge_attr, wb, mb):
    nreal = E // EB
    return pl.pallas_call(
        _eterm_body,
        grid=(EP // EB,),
        in_specs=[
            pl.BlockSpec((EB, ED), lambda i: (jnp.minimum(i, nreal - 1), 0)),
            pl.BlockSpec((ED, D), lambda i: (0, 0)),
            pl.BlockSpec((1, D), lambda i: (0, 0)),
        ],
        out_specs=pl.BlockSpec((EB, D), lambda i: (i, 0)),
        out_shape=jax.ShapeDtypeStruct((EP, D), jnp.float32),
    )(edge_attr, wb, mb)


def _sc_edge_body(p_hbm, et_hbm, src_hbm, dst_hbm, za_hbm, zd_hbm,
                  agg_hbm, deg_hbm,
                  agg_sh, deg_sh, sidx, didx, ones_v, pbuf, ebuf, gsem):
    core = lax.axis_index("c")
    sub = lax.axis_index("s")
    w = core * NS + sub
    stripe = NP // NS
    base = sub * stripe

    pltpu.sync_copy(za_hbm.at[pl.ds(base, stripe)], agg_sh.at[pl.ds(base, stripe)])
    pltpu.sync_copy(zd_hbm.at[pl.ds(base, stripe)], deg_sh.at[pl.ds(base, stripe)])

    def _fill(i, carry):
        ones_v[pl.ds(i * 16, 16)] = jnp.full((16,), 1.0, jnp.float32)
        return carry

    lax.fori_loop(0, CH // 16, _fill, 0)
    plsc.subcore_barrier()

    def _chunk(j, carry):
        pltpu.sync_copy(src_hbm.at[w, j], sidx)
        pltpu.sync_copy(dst_hbm.at[w, j], didx)
        pltpu.async_copy(p_hbm.at[sidx], pbuf, gsem).wait()
        pltpu.sync_copy(et_hbm.at[pl.ds((w * CPW + j) * CH, CH)], ebuf)

        def _row(r, c2):
            for cc in range(D // 16):
                sl = pl.ds(cc * 16, 16)
                pbuf[r, sl] = jnp.maximum(pbuf[r, sl] + ebuf[r, sl], 0.0)
            return c2

        lax.fori_loop(0, CH, _row, 0)
        pltpu.sync_copy(pbuf, agg_sh.at[didx], add=True)
        pltpu.sync_copy(ones_v, deg_sh.at[didx], add=True)
        return carry

    lax.fori_loop(0, CPW, _chunk, 0)

    plsc.subcore_barrier()
    pltpu.sync_copy(agg_sh.at[pl.ds(base, stripe)],
                    agg_hbm.at[core, pl.ds(base, stripe)])
    pltpu.sync_copy(deg_sh.at[pl.ds(base, stripe)],
                    deg_hbm.at[core, pl.ds(base, stripe)])


_sc_edge_built = None


def _sc_edge(*args):
    global _sc_edge_built
    if _sc_edge_built is None:
        mesh = plsc.VectorSubcoreMesh(core_axis_name="c", subcore_axis_name="s")
        _sc_edge_built = pl.kernel(
            _sc_edge_body,
            out_type=[
                jax.ShapeDtypeStruct((NC, NP, D), jnp.float32),
                jax.ShapeDtypeStruct((NC, NP), jnp.float32),
            ],
            mesh=mesh,
            scratch_types=[
                pltpu.VMEM_SHARED((NP, D), jnp.float32),
                pltpu.VMEM_SHARED((NP,), jnp.float32),
                pltpu.VMEM((CH,), jnp.int32),
                pltpu.VMEM((CH,), jnp.int32),
                pltpu.VMEM((CH,), jnp.float32),
                pltpu.VMEM((CH, D), jnp.float32),
                pltpu.VMEM((CH, D), jnp.float32),
                pltpu.SemaphoreType.DMA,
            ],
        )
    return _sc_edge_built(*args)


def _gru_body(out_ref, a0_ref, a1_ref, d0_ref, d1_ref,
              wi_ref, wh_ref, bi_ref, bh_ref, wt_ref,
              newout_ref, p_ref):
    deg = jnp.maximum(d0_ref[0] + d1_ref[0], 1.0)
    s = a0_ref[0] + a1_ref[0]
    agg = (s.reshape(ROWB // 128, 128, D) / deg[:, :, None]).reshape(ROWB, D)
    h = out_ref[...]
    gi = _dot(agg, wi_ref[...]) + bi_ref[...]
    gh = _dot(h, wh_ref[...]) + bh_ref[...]
    r = jax.nn.sigmoid(gi[:, 0:D] + gh[:, 0:D])
    z = jax.nn.sigmoid(gi[:, D:2 * D] + gh[:, D:2 * D])
    n = jnp.tanh(gi[:, 2 * D:3 * D] + r * gh[:, 2 * D:3 * D])
    ho = (1.0 - z) * n + z * h
    newout_ref[...] = ho
    p_ref[...] = _dot(ho, wt_ref[...])


def _gru(out, aggp, degr, wi, wh, bi, bh, wt):
    db = ROWB // 128
    return pl.pallas_call(
        _gru_body,
        grid=(NP // ROWB,),
        in_specs=[
            pl.BlockSpec((ROWB, D), lambda i: (i, 0)),
            pl.BlockSpec((1, ROWB, D), lambda i: (0, i, 0)),
            pl.BlockSpec((1, ROWB, D), lambda i: (1, i, 0)),
            pl.BlockSpec((1, db, 128), lambda i: (0, i, 0)),
            pl.BlockSpec((1, db, 128), lambda i: (1, i, 0)),
            pl.BlockSpec((D, 3 * D), lambda i: (0, 0)),
            pl.BlockSpec((D, 3 * D), lambda i: (0, 0)),
            pl.BlockSpec((1, 3 * D), lambda i: (0, 0)),
            pl.BlockSpec((1, 3 * D), lambda i: (0, 0)),
            pl.BlockSpec((D, D), lambda i: (0, 0)),
        ],
        out_specs=[
            pl.BlockSpec((ROWB, D), lambda i: (i, 0)),
            pl.BlockSpec((ROWB, D), lambda i: (i, 0)),
        ],
        out_shape=[
            jax.ShapeDtypeStruct((NP, D), jnp.float32),
            jax.ShapeDtypeStruct((NP, D), jnp.float32),
        ],
    )(out, aggp, aggp, degr, degr, wi, wh, bi, bh, wt)


def _s2s_body(out_ref, batch_ref, wi_ref, wh_ref, b_ref,
              w1_ref, b1_ref, w2_ref, b2_ref, w3_ref, b3_ref, v_ref):
    outv = out_ref[...]
    bidx = batch_ref[...]
    gids = lax.broadcasted_iota(jnp.int32, (1, B), 1)
    oneh = (bidx == gids).astype(jnp.float32)

    def step(t, carry):
        q_star, h, c = carry
        g = _dot(q_star, wi_ref[...]) + _dot(h, wh_ref[...]) + b_ref[...]
        ig = jax.nn.sigmoid(g[:, 0:D])
        fg = jax.nn.sigmoid(g[:, D:2 * D])
        gg = jnp.tanh(g[:, 2 * D:3 * D])
        og = jax.nn.sigmoid(g[:, 3 * D:4 * D])
        c2 = fg * c + ig * gg
        h2 = og * jnp.tanh(c2)
        qb = _dot(oneh, h2)
        e = jnp.sum(outv * qb, axis=1, keepdims=True)
        em = jnp.where(oneh > 0, e, -1e30)
        emax = jnp.max(em, axis=0, keepdims=True)
        emax_n = jnp.sum(oneh * emax, axis=1, keepdims=True)
        a = jnp.exp(e - emax_n)
        den = jnp.sum(oneh * a, axis=0, keepdims=True)
        den_n = jnp.sum(oneh * den, axis=1, keepdims=True)
        wgt = a / jnp.maximum(den_n, 1e-30)
        aw = oneh * wgt
        r_read = lax.dot_general(aw, outv, (((0,), (0,)), ((), ())),
                                 precision=_PREC,
                                 preferred_element_type=jnp.float32)
        q_star2 = jnp.concatenate([h2, r_read], axis=1)
        return (q_star2, h2, c2)

    init = (jnp.zeros((B, 2 * D), jnp.float32),
            jnp.zeros((B, D), jnp.float32),
            jnp.zeros((B, D), jnp.float32))
    q_star, _, _ = lax.fori_loop(0, 6, step, init)
    v = jnp.maximum(_dot(q_star, w1_ref[...]) + b1_ref[...], 0.0)
    v = jnp.maximum(_dot(v, w2_ref[...]) + b2_ref[...], 0.0)
    v_ref[...] = _dot(v, w3_ref[...]) + b3_ref[...]


def _s2s(out, batchp, wi, wh, b, w1, b1, w2, b2, w3p, b3p):
    return pl.pallas_call(
        _s2s_body,
        out_shape=jax.ShapeDtypeStruct((B, 128), jnp.float32),
    )(out, batchp, wi, wh, b, w1, b1, w2, b2, w3p, b3p)


def kernel(x, edge_index, edge_attr, batch,
           lin0_W, lin0_b, msg_W, msg_b,
           gru_Wi, gru_Wh, gru_bi, gru_bh,
           lstm_Wi, lstm_Wh, lstm_b,
           mlp_W1, mlp_b1, mlp_W2, mlp_b2, mlp_W3, mlp_b3):
    f32 = jnp.float32
    xp = jnp.pad(x, ((0, NP - N), (0, 0)))
    src = edge_index[0].astype(jnp.int32)
    dst = edge_index[1].astype(jnp.int32)
    pad_e = EP - E
    srcp = jnp.concatenate([src, jnp.zeros((pad_e,), jnp.int32)]).reshape(NW, CPW, CH)
    dstp = jnp.concatenate(
        [dst, N + (jnp.arange(pad_e, dtype=jnp.int32) % TRASH)]).reshape(NW, CPW, CH)
    w_top = msg_W[:D]
    w_bot = msg_W[D:]
    za = jnp.zeros((NP, D), f32)
    zd = jnp.zeros((NP,), f32)

    out, p = _lin0(xp, lin0_W, lin0_b.reshape(1, D), w_top)
    et = _eterm(edge_attr, w_bot, msg_b.reshape(1, D))

    for _ in range(3):
        aggp, degp = _sc_edge(p, et, srcp, dstp, za, zd)
        degr = degp.reshape(NC, NP // 128, 128)
        out, p = _gru(out, aggp, degr, gru_Wi, gru_Wh,
                      gru_bi.reshape(1, 3 * D), gru_bh.reshape(1, 3 * D), w_top)

    batchp = jnp.concatenate(
        [batch.astype(jnp.int32), jnp.full((NP - N,), B, jnp.int32)]).reshape(NP, 1)
    w3p = jnp.pad(mlp_W3, ((0, 0), (0, 127)))
    b3p = jnp.pad(mlp_b3.reshape(1, 1), ((0, 0), (0, 127)))
    v = _s2s(out, batchp, lstm_Wi, lstm_Wh, lstm_b.reshape(1, 4 * D),
             mlp_W1, mlp_b1.reshape(1, D), mlp_W2, mlp_b2.reshape(1, D),
             w3p, b3p)
    return v[:, :1]


# speedup vs baseline: 3.2994x; 1.1945x over previous
import functools

import jax
import jax.numpy as jnp
from jax import lax
from jax.experimental import pallas as pl
from jax.experimental.pallas import tpu as pltpu
from jax.experimental.pallas import tpu_sc as plsc

N = 10000
E = 160000
NF = 128
ED = 16
D = 128
B = 64

NP = 10240
NC = 2
NS = 16
NW = NC * NS
CH = 128
CPW = 40
EP = NW * CPW * CH
TRASH = NP - N
EB = 1280
ROWB = 2048

_PRECX = lax.Precision.HIGHEST


def _dot(a, b):
    return jnp.dot(a, b, preferred_element_type=jnp.float32)


def _lin0_body(x_ref, w0_ref, b0_ref, wt_ref, out_ref, p_ref):
    h = jnp.maximum(_dot(x_ref[...], w0_ref[...]) + b0_ref[...], 0.0)
    out_ref[...] = h
    p_ref[...] = _dot(h, wt_ref[...])


def _lin0(xp, w0, b0, wt):
    return pl.pallas_call(
        _lin0_body,
        grid=(NP // ROWB,),
        in_specs=[
            pl.BlockSpec((ROWB, NF), lambda i: (i, 0)),
            pl.BlockSpec((NF, D), lambda i: (0, 0)),
            pl.BlockSpec((1, D), lambda i: (0, 0)),
            pl.BlockSpec((D, D), lambda i: (0, 0)),
        ],
        out_specs=[
            pl.BlockSpec((ROWB, D), lambda i: (i, 0)),
            pl.BlockSpec((ROWB, D), lambda i: (i, 0)),
        ],
        out_shape=[
            jax.ShapeDtypeStruct((NP, D), jnp.float32),
            jax.ShapeDtypeStruct((NP, D), jnp.float32),
        ],
    )(xp, w0, b0, wt)


def _eterm_body(ea_ref, wb_ref, b_ref, out_ref):
    out_ref[...] = _dot(ea_ref[...], wb_ref[...]) + b_ref[...]


def _eterm(edge_attr, wb, mb):
    nreal = E // EB
    return pl.pallas_call(
        _eterm_body,
        grid=(EP // EB,),
        in_specs=[
            pl.BlockSpec((EB, ED), lambda i: (jnp.minimum(i, nreal - 1), 0)),
            pl.BlockSpec((ED, D), lambda i: (0, 0)),
            pl.BlockSpec((1, D), lambda i: (0, 0)),
        ],
        out_specs=pl.BlockSpec((EB, D), lambda i: (i, 0)),
        out_shape=jax.ShapeDtypeStruct((EP, D), jnp.float32),
    )(edge_attr, wb, mb)


def _sc_edge_body(p_hbm, et_hbm, idx_hbm, za_hbm, zd_hbm,
                  agg_hbm, deg_hbm,
                  agg_sh, deg_sh, ibuf0, ones_v,
                  pbuf0, ebuf0, gsem0, esem0, ssem0):
    core = lax.axis_index("c")
    sub = lax.axis_index("s")
    w = core * NS + sub
    stripe = NP // NS
    base = sub * stripe

    ibufs = (ibuf0,)
    pbufs = (pbuf0,)
    ebufs = (ebuf0,)
    gsems = (gsem0,)
    esems = (esem0,)
    ssems = (ssem0,)

    pltpu.sync_copy(za_hbm.at[pl.ds(base, stripe)], agg_sh.at[pl.ds(base, stripe)])
    pltpu.sync_copy(zd_hbm.at[pl.ds(base, stripe)], deg_sh.at[pl.ds(base, stripe)])

    def _fill(i, carry):
        ones_v[pl.ds(i * 16, 16)] = jnp.full((16,), 1.0, jnp.float32)
        return carry

    lax.fori_loop(0, CH // 16, _fill, 0)
    plsc.subcore_barrier()

    def _issue_in(j, b):
        pltpu.sync_copy(idx_hbm.at[w, j], ibufs[b])
        pltpu.async_copy(p_hbm.at[ibufs[b].at[0]], pbufs[b], gsems[b])
        pltpu.async_copy(et_hbm.at[pl.ds((w * CPW + j) * CH, CH)], ebufs[b],
                         esems[b])

    def _wait_in(b):
        pltpu.make_async_copy(p_hbm.at[ibufs[b].at[0]], pbufs[b],
                              gsems[b]).wait()
        pltpu.make_async_copy(et_hbm.at[pl.ds(0, CH)], ebufs[b],
                              esems[b]).wait()

    def _issue_out(j, b):
        pltpu.async_copy(pbufs[b], agg_sh.at[ibufs[b].at[1]], ssems[b],
                         add=True)
        pltpu.async_copy(ones_v, deg_sh.at[ibufs[b].at[1]], ssems[b],
                         add=True)

    def _wait_out(b):
        pltpu.make_async_copy(pbufs[b], agg_sh.at[ibufs[b].at[1]],
                              ssems[b]).wait()
        pltpu.make_async_copy(ones_v, deg_sh.at[ibufs[b].at[1]],
                              ssems[b]).wait()

    def _compute(b):
        pb = pbufs[b]
        eb = ebufs[b]

        def _row(r, c2):
            for cc in range(D // 16):
                sl = pl.ds(cc * 16, 16)
                pb[r, sl] = jnp.maximum(pb[r, sl] + eb[r, sl], 0.0)
            return c2

        lax.fori_loop(0, CH, _row, 0)

    def _body(j, carry):
        _issue_in(j, 0)
        _wait_in(0)
        _compute(0)
        _issue_out(j, 0)
        _wait_out(0)
        return carry

    lax.fori_loop(0, CPW, _body, 0)

    plsc.subcore_barrier()
    pltpu.sync_copy(agg_sh.at[pl.ds(base, stripe)],
                    agg_hbm.at[core, pl.ds(base, stripe)])
    pltpu.sync_copy(deg_sh.at[pl.ds(base, stripe)],
                    deg_hbm.at[core, pl.ds(base, stripe)])


_sc_edge_built = None


def _sc_edge(*args):
    global _sc_edge_built
    if _sc_edge_built is None:
        mesh = plsc.VectorSubcoreMesh(core_axis_name="c", subcore_axis_name="s")
        _sc_edge_built = pl.kernel(
            _sc_edge_body,
            out_type=[
                jax.ShapeDtypeStruct((NC, NP, D), jnp.float32),
                jax.ShapeDtypeStruct((NC, NP), jnp.float32),
            ],
            mesh=mesh,
            scratch_types=[
                pltpu.VMEM_SHARED((NP, D), jnp.float32),
                pltpu.VMEM_SHARED((NP,), jnp.float32),
                pltpu.VMEM((2, CH), jnp.int32),
                pltpu.VMEM((CH,), jnp.float32),
                pltpu.VMEM((CH, D), jnp.float32),
                pltpu.VMEM((CH, D), jnp.float32),
                pltpu.SemaphoreType.DMA,
                pltpu.SemaphoreType.DMA,
                pltpu.SemaphoreType.DMA,
            ],
        )
    return _sc_edge_built(*args)


def _gru_body(out_ref, a0_ref, a1_ref, d0_ref, d1_ref,
              wi_ref, wh_ref, bi_ref, bh_ref, wt_ref,
              newout_ref, p_ref):
    deg = jnp.maximum(d0_ref[0] + d1_ref[0], 1.0)
    s = a0_ref[0] + a1_ref[0]
    agg = (s.reshape(ROWB // 128, 128, D) / deg[:, :, None]).reshape(ROWB, D)
    h = out_ref[...]
    gi = _dot(agg, wi_ref[...]) + bi_ref[...]
    gh = _dot(h, wh_ref[...]) + bh_ref[...]
    r = jax.nn.sigmoid(gi[:, 0:D] + gh[:, 0:D])
    z = jax.nn.sigmoid(gi[:, D:2 * D] + gh[:, D:2 * D])
    n = jnp.tanh(gi[:, 2 * D:3 * D] + r * gh[:, 2 * D:3 * D])
    ho = (1.0 - z) * n + z * h
    newout_ref[...] = ho
    p_ref[...] = _dot(ho, wt_ref[...])


def _gru(out, aggp, degr, wi, wh, bi, bh, wt):
    db = ROWB // 128
    return pl.pallas_call(
        _gru_body,
        grid=(NP // ROWB,),
        in_specs=[
            pl.BlockSpec((ROWB, D), lambda i: (i, 0)),
            pl.BlockSpec((1, ROWB, D), lambda i: (0, i, 0)),
            pl.BlockSpec((1, ROWB, D), lambda i: (1, i, 0)),
            pl.BlockSpec((1, db, 128), lambda i: (0, i, 0)),
            pl.BlockSpec((1, db, 128), lambda i: (1, i, 0)),
            pl.BlockSpec((D, 3 * D), lambda i: (0, 0)),
            pl.BlockSpec((D, 3 * D), lambda i: (0, 0)),
            pl.BlockSpec((1, 3 * D), lambda i: (0, 0)),
            pl.BlockSpec((1, 3 * D), lambda i: (0, 0)),
            pl.BlockSpec((D, D), lambda i: (0, 0)),
        ],
        out_specs=[
            pl.BlockSpec((ROWB, D), lambda i: (i, 0)),
            pl.BlockSpec((ROWB, D), lambda i: (i, 0)),
        ],
        out_shape=[
            jax.ShapeDtypeStruct((NP, D), jnp.float32),
            jax.ShapeDtypeStruct((NP, D), jnp.float32),
        ],
    )(out, aggp, aggp, degr, degr, wi, wh, bi, bh, wt)


def _s2s_body(out_ref, batch_ref, wi_ref, wh_ref, b_ref,
              w1_ref, b1_ref, w2_ref, b2_ref, w3_ref, b3_ref, v_ref):
    outv = out_ref[...]
    bidx = batch_ref[...]
    gids = lax.broadcasted_iota(jnp.int32, (1, B), 1)
    oneh = (bidx == gids).astype(jnp.float32)

    def step(t, carry):
        q_star, h, c = carry
        g = _dot(q_star, wi_ref[...]) + _dot(h, wh_ref[...]) + b_ref[...]
        ig = jax.nn.sigmoid(g[:, 0:D])
        fg = jax.nn.sigmoid(g[:, D:2 * D])
        gg = jnp.tanh(g[:, 2 * D:3 * D])
        og = jax.nn.sigmoid(g[:, 3 * D:4 * D])
        c2 = fg * c + ig * gg
        h2 = og * jnp.tanh(c2)
        qb = jnp.dot(oneh, h2, precision=_PRECX,
                     preferred_element_type=jnp.float32)
        e = jnp.sum(outv * qb, axis=1, keepdims=True)
        em = jnp.where(oneh > 0, e, -1e30)
        emax = jnp.max(em, axis=0, keepdims=True)
        emax_n = jnp.sum(oneh * emax, axis=1, keepdims=True)
        a = jnp.exp(e - emax_n)
        den = jnp.sum(oneh * a, axis=0, keepdims=True)
        den_n = jnp.sum(oneh * den, axis=1, keepdims=True)
        wgt = a / jnp.maximum(den_n, 1e-30)
        aw = oneh * wgt
        r_read = lax.dot_general(aw, outv, (((0,), (0,)), ((), ())),
                                 precision=_PRECX,
                                 preferred_element_type=jnp.float32)
        q_star2 = jnp.concatenate([h2, r_read], axis=1)
        return (q_star2, h2, c2)

    init = (jnp.zeros((B, 2 * D), jnp.float32),
            jnp.zeros((B, D), jnp.float32),
            jnp.zeros((B, D), jnp.float32))
    q_star, _, _ = lax.fori_loop(0, 6, step, init)
    v = jnp.maximum(_dot(q_star, w1_ref[...]) + b1_ref[...], 0.0)
    v = jnp.maximum(_dot(v, w2_ref[...]) + b2_ref[...], 0.0)
    v_ref[...] = _dot(v, w3_ref[...]) + b3_ref[...]


def _s2s(out, batchp, wi, wh, b, w1, b1, w2, b2, w3p, b3p):
    return pl.pallas_call(
        _s2s_body,
        out_shape=jax.ShapeDtypeStruct((B, 128), jnp.float32),
    )(out, batchp, wi, wh, b, w1, b1, w2, b2, w3p, b3p)


def kernel(x, edge_index, edge_attr, batch,
           lin0_W, lin0_b, msg_W, msg_b,
           gru_Wi, gru_Wh, gru_bi, gru_bh,
           lstm_Wi, lstm_Wh, lstm_b,
           mlp_W1, mlp_b1, mlp_W2, mlp_b2, mlp_W3, mlp_b3):
    f32 = jnp.float32
    xp = jnp.pad(x, ((0, NP - N), (0, 0)))
    src = edge_index[0].astype(jnp.int32)
    dst = edge_index[1].astype(jnp.int32)
    pad_e = EP - E
    pad_iota = jnp.arange(pad_e, dtype=jnp.int32)
    srcp = jnp.concatenate([src, jnp.zeros((pad_e,), jnp.int32)]).reshape(NW, CPW, CH)
    dstp = jnp.concatenate([dst, N + pad_iota % TRASH]).reshape(NW, CPW, CH)
    idxp = jnp.stack([srcp, dstp], axis=2)
    w_top = msg_W[:D]
    w_bot = msg_W[D:]
    za = jnp.zeros((NP, D), f32)
    zd = jnp.zeros((NP,), f32)

    out, p = _lin0(xp, lin0_W, lin0_b.reshape(1, D), w_top)
    et = _eterm(edge_attr, w_bot, msg_b.reshape(1, D))

    for _ in range(3):
        aggp, degp = _sc_edge(p, et, idxp, za, zd)
        degr = degp.reshape(NC, NP // 128, 128)
        out, p = _gru(out, aggp, degr, gru_Wi, gru_Wh,
                      gru_bi.reshape(1, 3 * D), gru_bh.reshape(1, 3 * D), w_top)

    batchp = jnp.concatenate(
        [batch.astype(jnp.int32), jnp.full((NP - N,), B, jnp.int32)]).reshape(NP, 1)
    w3p = jnp.pad(mlp_W3, ((0, 0), (0, 127)))
    b3p = jnp.pad(mlp_b3.reshape(1, 1), ((0, 0), (0, 127)))
    v = _s2s(out, batchp, lstm_Wi, lstm_Wh, lstm_b.reshape(1, 4 * D),
             mlp_W1, mlp_b1.reshape(1, D), mlp_W2, mlp_b2.reshape(1, D),
             w3p, b3p)
    return v[:, :1]
